# baseline (speedup 1.0000x reference)
import functools

import jax
import jax.numpy as jnp
from jax import lax
from jax.experimental import pallas as pl
from jax.experimental.pallas import tpu as pltpu
from jax.experimental.pallas import tpu_sc as plsc

_N = 10000
_E = 2048
_DF = 128
_DE = 16
_DG = 8

_NC = 2
_NS = 16
_NW = _NC * _NS
_B = 2 * _E
_BPW = _B // _NW
_BPT = _E // _NS
_CH = 624
_CT = _N - _CH * _NS

@functools.cache
def _sc_kernels():
    mesh = plsc.VectorSubcoreMesh(
        core_axis_name="c", subcore_axis_name="s", num_cores=_NC)

    @functools.partial(
        pl.kernel,
        out_type=jax.ShapeDtypeStruct((_B, _DF), jnp.float32),
        mesh=mesh,
        scratch_types=[
            pltpu.VMEM((_BPW,), jnp.int32),
            pltpu.VMEM((_BPW, _DF), jnp.float32),
            pltpu.SemaphoreType.DMA,
        ],
    )
    def sc_gather(table_hbm, idx_hbm, out_hbm, idx_v, rows_v, sem):
        wid = lax.axis_index("s") * _NC + lax.axis_index("c")
        base = wid * _BPW
        pltpu.sync_copy(idx_hbm.at[pl.ds(base, _BPW)], idx_v)
        pltpu.async_copy(table_hbm.at[idx_v], rows_v, sem).wait()
        pltpu.sync_copy(rows_v, out_hbm.at[pl.ds(base, _BPW)])

    @functools.partial(
        pl.kernel,
        out_type=jax.ShapeDtypeStruct((_N, _DF), jnp.float32),
        mesh=mesh,
        scratch_types=[
            pltpu.VMEM_SHARED((_N, _DF), jnp.float32),
            pltpu.VMEM((_BPT,), jnp.int32),
            pltpu.VMEM((_BPT,), jnp.int32),
            pltpu.VMEM((_BPT, _DF), jnp.float32),
            pltpu.VMEM((_BPT, _DF), jnp.float32),
            pltpu.SemaphoreType.DMA,
            pltpu.SemaphoreType.DMA,
            pltpu.SemaphoreType.DMA,
            pltpu.SemaphoreType.DMA,
            pltpu.SemaphoreType.DMA,
            pltpu.SemaphoreType.DMA,
        ],
    )
    def sc_scatter(base_hbm, rows_hbm, idx_hbm, out_hbm,
                   acc_sh, idx_v0, idx_v1, rows_v0, rows_v1,
                   s0, s1, s2, s3, s4, s5):
        cid = lax.axis_index("c")
        sid = lax.axis_index("s")

        @pl.when(cid == 0)
        def _():
            coff = sid * _CH
            r0 = sid * (2 * _BPT)
            c_i0 = pltpu.async_copy(idx_hbm.at[pl.ds(r0, _BPT)], idx_v0, s0)
            c_i1 = pltpu.async_copy(idx_hbm.at[pl.ds(r0 + _BPT, _BPT)],
                                    idx_v1, s1)
            c_r0 = pltpu.async_copy(rows_hbm.at[pl.ds(r0, _BPT)], rows_v0, s2)
            c_r1 = pltpu.async_copy(rows_hbm.at[pl.ds(r0 + _BPT, _BPT)],
                                    rows_v1, s3)
            c_b = pltpu.async_copy(base_hbm.at[pl.ds(coff, _CH)],
                                   acc_sh.at[pl.ds(coff, _CH)], s4)
            c_o = pltpu.async_copy(base_hbm.at[pl.ds(coff, _CH)],
                                   out_hbm.at[pl.ds(coff, _CH)], s5)

            @pl.when(sid == _NS - 1)
            def _():
                pltpu.sync_copy(base_hbm.at[pl.ds(_CH * _NS, _CT)],
                                acc_sh.at[pl.ds(_CH * _NS, _CT)])
                pltpu.sync_copy(base_hbm.at[pl.ds(_CH * _NS, _CT)],
                                out_hbm.at[pl.ds(_CH * _NS, _CT)])

            c_i0.wait()
            c_i1.wait()
            c_r0.wait()
            c_r1.wait()
            c_b.wait()
            c_o.wait()
            plsc.subcore_barrier()
            pltpu.sync_copy(rows_v0, acc_sh.at[idx_v0], add=True)
            pltpu.sync_copy(rows_v1, acc_sh.at[idx_v1], add=True)
            plsc.subcore_barrier()
            g0 = pltpu.async_copy(acc_sh.at[idx_v0], rows_v0, s2)
            g1 = pltpu.async_copy(acc_sh.at[idx_v1], rows_v1, s3)
            g0.wait()
            g1.wait()
            w0 = pltpu.async_copy(rows_v0, out_hbm.at[idx_v0], s4)
            w1 = pltpu.async_copy(rows_v1, out_hbm.at[idx_v1], s5)
            w0.wait()
            w1.wait()

    return sc_gather, sc_scatter


_EB = 512


def _bf(x):
    return x.astype(jnp.bfloat16)


_KB = _E // _EB
_NB = 2000
_NBK = _N // _NB
_IN_E = _DE + 2 * _DF + _DG


def _fused_body(ein_ref, g_ref,
                w1_ref, b1_ref,
                w2_ref, b2_ref, wns_ref, wnr_ref,
                nodes_ref, wnn_ref, wng_ref, bn1_ref,
                eout_ref, rows_ref, base_ref, h1_scr):
    j = pl.program_id(0)

    for jj in range(_KB):
        @pl.when(j == jj)
        def _():
            acc = jnp.dot(ein_ref[...], _bf(w1_ref[...]),
                          preferred_element_type=jnp.float32)
            h1_scr[:, jj * _EB:(jj + 1) * _EB] = _bf(
                jnp.maximum(acc + b1_ref[...], 0.0))

    @pl.when(jnp.logical_and(j >= _KB, j < 2 * _KB))
    def _():
        acc = jnp.dot(h1_scr[...], _bf(w2_ref[...]),
                      preferred_element_type=jnp.float32)
        eb = jnp.maximum(acc + b2_ref[...], 0.0)
        eout_ref[...] = eb
        ebb = _bf(eb)
        es_p = jnp.dot(ebb, _bf(wns_ref[...]), preferred_element_type=jnp.float32)
        er_p = jnp.dot(ebb, _bf(wnr_ref[...]), preferred_element_type=jnp.float32)

        @pl.when(j == _KB)
        def _():
            rows_ref[:_E] = es_p
            rows_ref[_E:] = er_p

        @pl.when(j > _KB)
        def _():
            rows_ref[:_E] += es_p
            rows_ref[_E:] += er_p

    @pl.when(j >= 2 * _KB)
    def _():
        acc = jnp.dot(_bf(nodes_ref[...]), _bf(wnn_ref[...]),
                      preferred_element_type=jnp.float32)
        acc += jnp.dot(g_ref[...], wng_ref[...],
                       preferred_element_type=jnp.float32)
        base_ref[...] = acc + bn1_ref[...]


def _fused_tc(ein, globals_, w1, b1,
              w2, b2, wns, wnr, nodes, wnn, wng, bn1):
    def _jb(j):
        return jnp.clip(j - _KB, 0, _KB - 1)

    def _jn(j):
        return jnp.clip(j - 2 * _KB, 0, _NBK - 1)

    grid = (2 * _KB + _NBK,)
    return pl.pallas_call(
        _fused_body,
        grid=grid,
        in_specs=[
            pl.BlockSpec((_E, _IN_E), lambda j: (0, 0)),
            pl.BlockSpec((1, _DG), lambda j: (0, 0)),
            pl.BlockSpec((_IN_E, _EB), lambda j: (0, jnp.minimum(j, _KB - 1))),
            pl.BlockSpec((1, _EB), lambda j: (0, jnp.minimum(j, _KB - 1))),
            pl.BlockSpec((_E, _EB), lambda j: (0, _jb(j))),
            pl.BlockSpec((1, _EB), lambda j: (0, _jb(j))),
            pl.BlockSpec((_EB, _DF), lambda j: (_jb(j), 0)),
            pl.BlockSpec((_EB, _DF), lambda j: (_jb(j), 0)),
            pl.BlockSpec((_NB, _DF), lambda j: (_jn(j), 0)),
            pl.BlockSpec((_DF, _DF), lambda j: (0, 0)),
            pl.BlockSpec((_DG, _DF), lambda j: (0, 0)),
            pl.BlockSpec((1, _DF), lambda j: (0, 0)),
        ],
        out_specs=[
            pl.BlockSpec((_E, _EB), lambda j: (0, _jb(j))),
            pl.BlockSpec((2 * _E, _DF), lambda j: (0, 0)),
            pl.BlockSpec((_NB, _DF), lambda j: (_jn(j), 0)),
        ],
        out_shape=[
            jax.ShapeDtypeStruct((_E, _E), jnp.float32),
            jax.ShapeDtypeStruct((2 * _E, _DF), jnp.float32),
            jax.ShapeDtypeStruct((_N, _DF), jnp.float32),
        ],
        scratch_shapes=[pltpu.VMEM((_E, _E), jnp.bfloat16)],
    )(ein, globals_, w1, b1,
      w2, b2, wns, wnr, nodes, wnn, wng, bn1)


_TB = 2000


def _node_body(hp_ref, w2_ref, b2_ref, out_ref):
    h = jnp.maximum(hp_ref[...], 0.0)
    acc = jnp.dot(_bf(h), _bf(w2_ref[...]), preferred_element_type=jnp.float32)
    out_ref[...] = jnp.maximum(acc + b2_ref[...], 0.0)


def _node_tail(hp, wn2, bn2):
    grid = (_N // _TB,)
    return pl.pallas_call(
        _node_body,
        grid=grid,
        in_specs=[
            pl.BlockSpec((_TB, _DF), lambda i: (i, 0)),
            pl.BlockSpec((_DF, _DF), lambda i: (0, 0)),
            pl.BlockSpec((1, _DF), lambda i: (0, 0)),
        ],
        out_specs=pl.BlockSpec((_TB, _DF), lambda i: (i, 0)),
        out_shape=jax.ShapeDtypeStruct((_N, _DF), jnp.float32),
    )(hp, wn2, bn2)


def kernel(nodes, edges, receivers, senders, globals_, n_node, n_edge,
           W_e1, b_e1, W_e2, b_e2, W_n1, b_n1, W_n2, b_n2):
    sc_gather, sc_scatter = _sc_kernels()
    idx = jnp.concatenate([senders, receivers])
    gathered = sc_gather(nodes, idx)

    ein = jnp.concatenate(
        [edges, gathered[:_E], gathered[_E:],
         jnp.broadcast_to(globals_[0], (_E, _DG))],
        axis=1).astype(jnp.bfloat16)
    wns = W_n1[_DF:_DF + _E]
    wnr = W_n1[_DF + _E:_DF + 2 * _E]
    edges_new, rows, base = _fused_tc(
        ein, globals_, W_e1, b_e1.reshape(1, -1),
        W_e2, b_e2.reshape(1, -1), wns, wnr,
        nodes, W_n1[:_DF], W_n1[_DF + 2 * _E:], b_n1.reshape(1, -1))
    hpre = sc_scatter(base, rows, idx)
    nodes_new = _node_tail(hpre, W_n2, b_n2.reshape(1, -1))

    return (nodes_new, edges_new, receivers, senders, globals_, n_node, n_edge)


# speedup vs baseline: 2.4838x; 2.4838x over previous
import functools

import jax
import jax.numpy as jnp
from jax import lax
from jax.experimental import pallas as pl
from jax.experimental.pallas import tpu as pltpu
from jax.experimental.pallas import tpu_sc as plsc

_N = 10000
_E = 2048
_DF = 128
_DE = 16
_DG = 8

_NC = 2
_NS = 16
_NW = _NC * _NS
_B = 2 * _E
_BPW = _B // _NW
_BPT = _E // _NS
_CH = 624
_CT = _N - _CH * _NS

@functools.cache
def _sc_kernels():
    mesh = plsc.VectorSubcoreMesh(
        core_axis_name="c", subcore_axis_name="s", num_cores=_NC)

    @functools.partial(
        pl.kernel,
        out_type=jax.ShapeDtypeStruct((_B, _DF), jnp.float32),
        mesh=mesh,
        scratch_types=[
            pltpu.VMEM((_BPW,), jnp.int32),
            pltpu.VMEM((_BPW, _DF), jnp.float32),
            pltpu.SemaphoreType.DMA,
        ],
    )
    def sc_gather(table_hbm, idx_hbm, out_hbm, idx_v, rows_v, sem):
        wid = lax.axis_index("s") * _NC + lax.axis_index("c")
        base = wid * _BPW
        pltpu.sync_copy(idx_hbm.at[pl.ds(base, _BPW)], idx_v)
        pltpu.async_copy(table_hbm.at[idx_v], rows_v, sem).wait()
        pltpu.sync_copy(rows_v, out_hbm.at[pl.ds(base, _BPW)])

    @functools.partial(
        pl.kernel,
        out_type=(
            jax.ShapeDtypeStruct((_N, _DF), jnp.float32),
            jax.ShapeDtypeStruct((_N, _DF), jnp.float32),
        ),
        mesh=mesh,
        scratch_types=[
            pltpu.VMEM_SHARED((_N, _DF), jnp.float32),
            pltpu.VMEM((_BPT,), jnp.int32),
            pltpu.VMEM((_BPT, _DF), jnp.float32),
            pltpu.SemaphoreType.DMA,
            pltpu.SemaphoreType.DMA,
            pltpu.SemaphoreType.DMA,
        ],
    )
    def sc_scatter(basehalf_hbm, rows_hbm, idx_hbm, out0_hbm, out1_hbm,
                   acc_sh, idx_v, rows_v, sem_i, sem_r, sem_b):
        cid = lax.axis_index("c")
        sid = lax.axis_index("s")
        coff = sid * _CH
        roff = cid * _E + sid * _BPT
        c_i = pltpu.async_copy(idx_hbm.at[pl.ds(roff, _BPT)], idx_v, sem_i)
        c_r = pltpu.async_copy(rows_hbm.at[pl.ds(roff, _BPT)], rows_v, sem_r)
        c_b = pltpu.async_copy(basehalf_hbm.at[pl.ds(coff, _CH)],
                               acc_sh.at[pl.ds(coff, _CH)], sem_b)

        @pl.when(sid == _NS - 1)
        def _():
            pltpu.sync_copy(basehalf_hbm.at[pl.ds(_CH * _NS, _CT)],
                            acc_sh.at[pl.ds(_CH * _NS, _CT)])

        c_i.wait()
        c_r.wait()
        c_b.wait()
        plsc.subcore_barrier()
        pltpu.sync_copy(rows_v, acc_sh.at[idx_v], add=True)
        plsc.subcore_barrier()

        @pl.when(cid == 0)
        def _():
            pltpu.sync_copy(acc_sh.at[pl.ds(coff, _CH)],
                            out0_hbm.at[pl.ds(coff, _CH)])

            @pl.when(sid == _NS - 1)
            def _():
                pltpu.sync_copy(acc_sh.at[pl.ds(_CH * _NS, _CT)],
                                out0_hbm.at[pl.ds(_CH * _NS, _CT)])

        @pl.when(cid == 1)
        def _():
            pltpu.sync_copy(acc_sh.at[pl.ds(coff, _CH)],
                            out1_hbm.at[pl.ds(coff, _CH)])

            @pl.when(sid == _NS - 1)
            def _():
                pltpu.sync_copy(acc_sh.at[pl.ds(_CH * _NS, _CT)],
                                out1_hbm.at[pl.ds(_CH * _NS, _CT)])

    return sc_gather, sc_scatter


_EB = 512


def _bf(x):
    return x.astype(jnp.bfloat16)


_KB = _E // _EB
_NB = 2000
_NBK = _N // _NB
_IN_E = _DE + 2 * _DF + _DG


def _fused_body(ein_ref, g_ref,
                w1_ref, b1_ref,
                w2_ref, b2_ref, wns_ref, wnr_ref,
                nodes_ref, wnn_ref, wng_ref, bn1_ref,
                eout_ref, rows_ref, base_ref, h1_scr):
    j = pl.program_id(0)

    for jj in range(_KB):
        @pl.when(j == jj)
        def _():
            acc = jnp.dot(ein_ref[...], _bf(w1_ref[...]),
                          preferred_element_type=jnp.float32)
            h1_scr[:, jj * _EB:(jj + 1) * _EB] = _bf(
                jnp.maximum(acc + b1_ref[...], 0.0))

    @pl.when(jnp.logical_and(j >= _KB, j < 2 * _KB))
    def _():
        acc = jnp.dot(h1_scr[...], _bf(w2_ref[...]),
                      preferred_element_type=jnp.float32)
        eb = jnp.maximum(acc + b2_ref[...], 0.0)
        eout_ref[...] = eb
        ebb = _bf(eb)
        es_p = jnp.dot(ebb, _bf(wns_ref[...]), preferred_element_type=jnp.float32)
        er_p = jnp.dot(ebb, _bf(wnr_ref[...]), preferred_element_type=jnp.float32)

        @pl.when(j == _KB)
        def _():
            rows_ref[:_E] = es_p
            rows_ref[_E:] = er_p

        @pl.when(j > _KB)
        def _():
            rows_ref[:_E] += es_p
            rows_ref[_E:] += er_p

    @pl.when(j >= 2 * _KB)
    def _():
        acc = jnp.dot(_bf(nodes_ref[...]), _bf(wnn_ref[...]),
                      preferred_element_type=jnp.float32)
        acc += jnp.dot(g_ref[...], wng_ref[...],
                       preferred_element_type=jnp.float32)
        base_ref[...] = 0.5 * (acc + bn1_ref[...])


def _fused_tc(ein, globals_, w1, b1,
              w2, b2, wns, wnr, nodes, wnn, wng, bn1):
    def _jb(j):
        return jnp.clip(j - _KB, 0, _KB - 1)

    def _jn(j):
        return jnp.clip(j - 2 * _KB, 0, _NBK - 1)

    grid = (2 * _KB + _NBK,)
    return pl.pallas_call(
        _fused_body,
        grid=grid,
        in_specs=[
            pl.BlockSpec((_E, _IN_E), lambda j: (0, 0)),
            pl.BlockSpec((1, _DG), lambda j: (0, 0)),
            pl.BlockSpec((_IN_E, _EB), lambda j: (0, jnp.minimum(j, _KB - 1))),
            pl.BlockSpec((1, _EB), lambda j: (0, jnp.minimum(j, _KB - 1))),
            pl.BlockSpec((_E, _EB), lambda j: (0, _jb(j))),
            pl.BlockSpec((1, _EB), lambda j: (0, _jb(j))),
            pl.BlockSpec((_EB, _DF), lambda j: (_jb(j), 0)),
            pl.BlockSpec((_EB, _DF), lambda j: (_jb(j), 0)),
            pl.BlockSpec((_NB, _DF), lambda j: (_jn(j), 0)),
            pl.BlockSpec((_DF, _DF), lambda j: (0, 0)),
            pl.BlockSpec((_DG, _DF), lambda j: (0, 0)),
            pl.BlockSpec((1, _DF), lambda j: (0, 0)),
        ],
        out_specs=[
            pl.BlockSpec((_E, _EB), lambda j: (0, _jb(j))),
            pl.BlockSpec((2 * _E, _DF), lambda j: (0, 0)),
            pl.BlockSpec((_NB, _DF), lambda j: (_jn(j), 0)),
        ],
        out_shape=[
            jax.ShapeDtypeStruct((_E, _E), jnp.float32),
            jax.ShapeDtypeStruct((2 * _E, _DF), jnp.float32),
            jax.ShapeDtypeStruct((_N, _DF), jnp.float32),
        ],
        scratch_shapes=[pltpu.VMEM((_E, _E), jnp.bfloat16)],
    )(ein, globals_, w1, b1,
      w2, b2, wns, wnr, nodes, wnn, wng, bn1)


_TB = 2000


def _node_body(p0_ref, p1_ref, w2_ref, b2_ref, out_ref):
    h = jnp.maximum(p0_ref[...] + p1_ref[...], 0.0)
    acc = jnp.dot(_bf(h), _bf(w2_ref[...]), preferred_element_type=jnp.float32)
    out_ref[...] = jnp.maximum(acc + b2_ref[...], 0.0)


def _node_tail(p0, p1, wn2, bn2):
    grid = (_N // _TB,)
    return pl.pallas_call(
        _node_body,
        grid=grid,
        in_specs=[
            pl.BlockSpec((_TB, _DF), lambda i: (i, 0)),
            pl.BlockSpec((_TB, _DF), lambda i: (i, 0)),
            pl.BlockSpec((_DF, _DF), lambda i: (0, 0)),
            pl.BlockSpec((1, _DF), lambda i: (0, 0)),
        ],
        out_specs=pl.BlockSpec((_TB, _DF), lambda i: (i, 0)),
        out_shape=jax.ShapeDtypeStruct((_N, _DF), jnp.float32),
    )(p0, p1, wn2, bn2)


def kernel(nodes, edges, receivers, senders, globals_, n_node, n_edge,
           W_e1, b_e1, W_e2, b_e2, W_n1, b_n1, W_n2, b_n2):
    sc_gather, sc_scatter = _sc_kernels()
    idx = jnp.concatenate([senders, receivers])
    gathered = sc_gather(nodes, idx)

    ein = jnp.concatenate(
        [edges, gathered[:_E], gathered[_E:],
         jnp.broadcast_to(globals_[0], (_E, _DG))],
        axis=1).astype(jnp.bfloat16)
    wns = W_n1[_DF:_DF + _E]
    wnr = W_n1[_DF + _E:_DF + 2 * _E]
    edges_new, rows, basehalf = _fused_tc(
        ein, globals_, W_e1, b_e1.reshape(1, -1),
        W_e2, b_e2.reshape(1, -1), wns, wnr,
        nodes, W_n1[:_DF], W_n1[_DF + 2 * _E:], b_n1.reshape(1, -1))
    p0, p1 = sc_scatter(basehalf, rows, idx)
    nodes_new = _node_tail(p0, p1, W_n2, b_n2.reshape(1, -1))

    return (nodes_new, edges_new, receivers, senders, globals_, n_node, n_edge)
